# baseline (speedup 1.0000x reference)
import functools

import jax
import jax.numpy as jnp
from jax import lax
from jax.experimental import pallas as pl
from jax.experimental.pallas import tpu as pltpu
from jax.experimental.pallas import tpu_sc as plsc

N = 10000
D = 128
E = 320000
NC = 2
NS = 16
NW = NC * NS
EPW = E // NW
C = 80
NCHUNK = EPW // C
NP = NS * 640
RPS = NP // NS
NPA = NS * 632
RPA = NPA // NS
NBUF = 4
LAST = NCHUNK - 1

_mesh = plsc.VectorSubcoreMesh(core_axis_name="c", subcore_axis_name="s")


@functools.partial(
    pl.kernel,
    out_type=(jax.ShapeDtypeStruct((NP,), jnp.float32),
              jax.ShapeDtypeStruct((NP,), jnp.float32)),
    mesh=_mesh,
    scratch_types=[
        pltpu.VMEM_SHARED((NP,), jnp.float32),
        pltpu.VMEM((NCHUNK, C), jnp.int32),
        pltpu.VMEM((C,), jnp.float32),
        pltpu.VMEM((RPS,), jnp.float32),
        pltpu.SemaphoreType.DMA,
        pltpu.SemaphoreType.DMA,
        pltpu.SemaphoreType.DMA,
        pltpu.SemaphoreType.DMA,
    ],
)
def _deg_kernel(dstr_hbm, ones_hbm, out0_hbm, out1_hbm,
                deg_sp, dsts_v, ones_v, buf_v, s0, s1, s2, s3):
    c = lax.axis_index("c")
    s = lax.axis_index("s")
    w = c * NS + s
    sems = [s0, s1, s2, s3]
    pltpu.sync_copy(ones_hbm, buf_v)
    pltpu.sync_copy(ones_hbm.at[pl.ds(0, C)], ones_v)
    pltpu.sync_copy(dstr_hbm.at[w], dsts_v)
    pltpu.sync_copy(buf_v, deg_sp.at[pl.ds(s * RPS, RPS)])
    plsc.subcore_barrier()

    def swait(sem):
        pltpu.make_async_copy(ones_v, deg_sp.at[dsts_v.at[0]], sem).wait()

    def body(j, carry):
        for b in range(NBUF):
            @pl.when(j > 0)
            def _():
                swait(sems[b])
            pltpu.async_copy(ones_v, deg_sp.at[dsts_v.at[j * NBUF + b]],
                             sems[b], add=True)
        return carry

    lax.fori_loop(0, NCHUNK // NBUF, body, 0)
    swait(sems[0])
    pltpu.async_copy(ones_v, deg_sp.at[dsts_v.at[NCHUNK - 1]], s0, add=True)
    for b in range(NBUF):
        swait(sems[b])
    plsc.subcore_barrier()
    pltpu.sync_copy(deg_sp.at[pl.ds(s * RPS, RPS)], buf_v)
    @pl.when(c == 0)
    def _():
        pltpu.sync_copy(buf_v, out0_hbm.at[pl.ds(s * RPS, RPS)])
    @pl.when(c == 1)
    def _():
        pltpu.sync_copy(buf_v, out1_hbm.at[pl.ds(s * RPS, RPS)])


@functools.partial(
    pl.kernel,
    out_type=(jax.ShapeDtypeStruct((NPA, D), jnp.float32),
              jax.ShapeDtypeStruct((NPA, D), jnp.float32)),
    mesh=_mesh,
    scratch_types=[
        pltpu.VMEM_SHARED((NPA, D), jnp.float32),
        [pltpu.VMEM((C, D), jnp.float32)] * NBUF,
        [[pltpu.VMEM((2, C), jnp.int32)] * 2] * NBUF,
        [pltpu.SemaphoreType.DMA] * NBUF,
        [pltpu.SemaphoreType.DMA] * NBUF,
        [pltpu.SemaphoreType.DMA] * NBUF,
        pltpu.SemaphoreType.DMA,
    ],
)
def _agg_kernel(g_hbm, srcf_hbm, dstf_hbm, zeros_hbm, out0_hbm, out1_hbm,
                acc_sp, rows, idx, isems, gsems, ssems, zsem):
    c = lax.axis_index("c")
    s = lax.axis_index("s")
    w = c * NS + s

    def iissue(i, b, p):
        off = pl.multiple_of(w * EPW + i * C, C)
        pltpu.async_copy(srcf_hbm.at[pl.ds(off, C)], idx[b][p].at[0],
                         isems[b])
        pltpu.async_copy(dstf_hbm.at[pl.ds(off, C)], idx[b][p].at[1],
                         isems[b])

    def iwait(b):
        pltpu.make_async_copy(srcf_hbm.at[pl.ds(0, C)], idx[b][0].at[0],
                              isems[b]).wait()
        pltpu.make_async_copy(srcf_hbm.at[pl.ds(0, C)], idx[b][0].at[1],
                              isems[b]).wait()

    def gissue(i, b, p):
        pltpu.async_copy(g_hbm.at[idx[b][p].at[0]], rows[b], gsems[b])

    def gwait(b):
        pltpu.make_async_copy(g_hbm.at[idx[0][0].at[0]], rows[b],
                              gsems[b]).wait()

    def sissue(i, b, p):
        pltpu.async_copy(rows[b], acc_sp.at[idx[b][p].at[1]], ssems[b],
                         add=True)

    def swait(b):
        pltpu.make_async_copy(rows[b], acc_sp.at[idx[0][0].at[1]],
                              ssems[b]).wait()

    def service(i):
        b, p = i % NBUF, (i // NBUF) % 2
        b1, p1 = (i + 1) % NBUF, ((i + 1) // NBUF) % 2
        if i + 1 <= LAST:
            iwait(b1)
            if i >= 3:
                swait(b1)
            gissue(i + 1, b1, p1)
        gwait(b)
        sissue(i, b, p)
        if i + 4 <= LAST:
            iissue(i + 4, b, 1 - p)

    for b in range(NBUF):
        iissue(b, b, 0)
    pltpu.sync_copy(zeros_hbm, rows[0])
    for j in range(8):
        sz = C if j < 7 else RPA - 7 * C
        pltpu.async_copy(rows[0].at[pl.ds(0, sz)],
                         acc_sp.at[pl.ds(s * RPA + j * C, sz)], zsem)
    for j in range(8):
        sz = C if j < 7 else RPA - 7 * C
        pltpu.make_async_copy(rows[0].at[pl.ds(0, sz)],
                              acc_sp.at[pl.ds(0, sz)], zsem).wait()
    plsc.subcore_barrier()
    iwait(0)
    gissue(0, 0, 0)
    for i in range(8):
        service(i)

    def body(j, carry):
        i0 = j * 8
        for k in range(8):
            i = i0 + k
            b, p = k % NBUF, (k // NBUF) % 2
            b1 = (k + 1) % NBUF
            p1 = ((k + 1) // NBUF) % 2
            iwait(b1)
            swait(b1)
            gissue(i + 1, b1, p1)
            gwait(b)
            sissue(i, b, p)
            iissue(i + 4, b, 1 - p)
        return carry

    lax.fori_loop(1, 15, body, 0)

    for i in range(120, NCHUNK):
        service(i)
    for b in [1, 2, 3, 0]:
        swait(b)
    plsc.subcore_barrier()

    def dread(j, b):
        sz = C if j < 7 else RPA - 7 * C
        pltpu.async_copy(acc_sp.at[pl.ds(s * RPA + j * C, sz)],
                         rows[b].at[pl.ds(0, sz)], gsems[b])

    def dwrite(j, b):
        sz = C if j < 7 else RPA - 7 * C
        pltpu.make_async_copy(acc_sp.at[pl.ds(0, sz)],
                              rows[b].at[pl.ds(0, sz)], gsems[b]).wait()
        @pl.when(c == 0)
        def _():
            pltpu.async_copy(rows[b].at[pl.ds(0, sz)],
                             out0_hbm.at[pl.ds(s * RPA + j * C, sz)],
                             ssems[b])
        @pl.when(c == 1)
        def _():
            pltpu.async_copy(rows[b].at[pl.ds(0, sz)],
                             out1_hbm.at[pl.ds(s * RPA + j * C, sz)],
                             ssems[b])

    def dwwait(j, b):
        sz = C if j < 7 else RPA - 7 * C
        pltpu.make_async_copy(rows[b].at[pl.ds(0, sz)],
                              out0_hbm.at[pl.ds(0, sz)], ssems[b]).wait()

    dread(0, 0)
    dread(1, 1)
    for j in range(8):
        b = j % 2
        dwrite(j, b)
        if j + 2 < 8:
            dwwait(j, b)
            dread(j + 2, b)
    dwwait(6, 0)
    dwwait(7, 1)


_BLK = 10240
_GRID = NP // _BLK


def _row_spec():
    return pl.BlockSpec((_BLK, D), lambda i: (i, 0))


def _col_spec():
    return pl.BlockSpec((_BLK, 1), lambda i: (i, 0))


def _full_spec():
    return pl.BlockSpec((D, D), lambda i: (0, 0))


def _bias_spec():
    return pl.BlockSpec((1, D), lambda i: (0, 0))


def _tc1_body(d0_ref, d1_ref, x_ref, w1_ref, g_ref, dinv_ref):
    deg = d0_ref[...] + d1_ref[...] - 1.0
    dinv = lax.rsqrt(deg)
    h = jnp.dot(x_ref[...], w1_ref[...], preferred_element_type=jnp.float32)
    g_ref[...] = h * dinv
    dinv_ref[...] = dinv


_tc1 = pl.pallas_call(
    _tc1_body,
    grid=(_GRID,),
    in_specs=[_col_spec(), _col_spec(), _row_spec(), _full_spec()],
    out_specs=[_row_spec(), _col_spec()],
    out_shape=(jax.ShapeDtypeStruct((NP, D), jnp.float32),
               jax.ShapeDtypeStruct((NP, 1), jnp.float32)),
)


def _tc2_body(p0_ref, p1_ref, g1_ref, dinv_ref, b1_ref, w2_ref, g2_ref):
    dinv = dinv_ref[...]
    a = dinv * (p0_ref[...] + p1_ref[...] + g1_ref[...]) + b1_ref[...]
    t = 0.5 * a * (1.0 + lax.erf(a * 0.7071067811865476))
    g2_ref[...] = jnp.dot(t, w2_ref[...],
                          preferred_element_type=jnp.float32) * dinv


_tc2 = pl.pallas_call(
    _tc2_body,
    grid=(_GRID,),
    in_specs=[_row_spec(), _row_spec(), _row_spec(), _col_spec(),
              _bias_spec(), _full_spec()],
    out_specs=_row_spec(),
    out_shape=jax.ShapeDtypeStruct((NP, D), jnp.float32),
)


def _tc3_body(q0_ref, q1_ref, g2_ref, dinv_ref, b2_ref, out_ref):
    out_ref[...] = (dinv_ref[...] * (q0_ref[...] + q1_ref[...] + g2_ref[...])
                    + b2_ref[...])


_tc3 = pl.pallas_call(
    _tc3_body,
    grid=(_GRID,),
    in_specs=[_row_spec(), _row_spec(), _row_spec(), _col_spec(),
              _bias_spec()],
    out_specs=_row_spec(),
    out_shape=jax.ShapeDtypeStruct((N, D), jnp.float32),
)


def kernel(x, edge_index, W1, b1, W2, b2):
    dst_r = edge_index[1].reshape(NW, NCHUNK, C)
    src_f = edge_index[0]
    dst_f = edge_index[1]
    ones_c = jnp.ones((RPS,), jnp.float32)
    zeros_c = jnp.zeros((C, D), jnp.float32)

    d0, d1 = _deg_kernel(dst_r, ones_c)
    g1, dinv = _tc1(d0.reshape(NP, 1), d1.reshape(NP, 1), x, W1)
    p0, p1 = _agg_kernel(g1, src_f, dst_f, zeros_c)
    g2 = _tc2(p0, p1, g1, dinv, b1.reshape(1, D), W2)
    q0, q1 = _agg_kernel(g2, src_f, dst_f, zeros_c)
    return _tc3(q0, q1, g2, dinv, b2.reshape(1, D))


# speedup vs baseline: 1.0098x; 1.0098x over previous
import functools

import jax
import jax.numpy as jnp
from jax import lax
from jax.experimental import pallas as pl
from jax.experimental.pallas import tpu as pltpu
from jax.experimental.pallas import tpu_sc as plsc

N = 10000
D = 128
E = 320000
NC = 2
NS = 16
NW = NC * NS
EPW = E // NW
C = 80
NCHUNK = EPW // C
NP = NS * 640
RPS = NP // NS
NPA = NS * 632
RPA = NPA // NS
NBUF = 4
LAST = NCHUNK - 1

_mesh = plsc.VectorSubcoreMesh(core_axis_name="c", subcore_axis_name="s")


@functools.partial(
    pl.kernel,
    out_type=(jax.ShapeDtypeStruct((NP,), jnp.float32),
              jax.ShapeDtypeStruct((NP,), jnp.float32)),
    mesh=_mesh,
    scratch_types=[
        pltpu.VMEM_SHARED((NP,), jnp.float32),
        pltpu.VMEM((NCHUNK, C), jnp.int32),
        pltpu.VMEM((C,), jnp.float32),
        pltpu.VMEM((RPS,), jnp.float32),
        pltpu.SemaphoreType.DMA,
        pltpu.SemaphoreType.DMA,
        pltpu.SemaphoreType.DMA,
        pltpu.SemaphoreType.DMA,
    ],
)
def _deg_kernel(dstr_hbm, ones_hbm, out0_hbm, out1_hbm,
                deg_sp, dsts_v, ones_v, buf_v, s0, s1, s2, s3):
    c = lax.axis_index("c")
    s = lax.axis_index("s")
    w = c * NS + s
    sems = [s0, s1, s2, s3]
    pltpu.sync_copy(ones_hbm, buf_v)
    pltpu.sync_copy(ones_hbm.at[pl.ds(0, C)], ones_v)
    pltpu.sync_copy(dstr_hbm.at[w], dsts_v)
    pltpu.sync_copy(buf_v, deg_sp.at[pl.ds(s * RPS, RPS)])
    plsc.subcore_barrier()

    def swait(sem):
        pltpu.make_async_copy(ones_v, deg_sp.at[dsts_v.at[0]], sem).wait()

    def body(j, carry):
        for b in range(NBUF):
            @pl.when(j > 0)
            def _():
                swait(sems[b])
            pltpu.async_copy(ones_v, deg_sp.at[dsts_v.at[j * NBUF + b]],
                             sems[b], add=True)
        return carry

    lax.fori_loop(0, NCHUNK // NBUF, body, 0)
    swait(sems[0])
    pltpu.async_copy(ones_v, deg_sp.at[dsts_v.at[NCHUNK - 1]], s0, add=True)
    for b in range(NBUF):
        swait(sems[b])
    plsc.subcore_barrier()
    pltpu.sync_copy(deg_sp.at[pl.ds(s * RPS, RPS)], buf_v)
    @pl.when(c == 0)
    def _():
        pltpu.sync_copy(buf_v, out0_hbm.at[pl.ds(s * RPS, RPS)])
    @pl.when(c == 1)
    def _():
        pltpu.sync_copy(buf_v, out1_hbm.at[pl.ds(s * RPS, RPS)])


@functools.partial(
    pl.kernel,
    out_type=(jax.ShapeDtypeStruct((NPA, D), jnp.float32),
              jax.ShapeDtypeStruct((NPA, D), jnp.float32)),
    mesh=_mesh,
    scratch_types=[
        pltpu.VMEM_SHARED((NPA, D), jnp.float32),
        [pltpu.VMEM((C, D), jnp.float32)] * NBUF,
        [[pltpu.VMEM((2, C), jnp.int32)] * 2] * NBUF,
        [pltpu.SemaphoreType.DMA] * NBUF,
        [pltpu.SemaphoreType.DMA] * NBUF,
        [pltpu.SemaphoreType.DMA] * NBUF,
        pltpu.SemaphoreType.DMA,
    ],
)
def _agg_kernel(g_hbm, srcf_hbm, dstf_hbm, zeros_hbm, out0_hbm, out1_hbm,
                acc_sp, rows, idx, isems, gsems, ssems, zsem):
    c = lax.axis_index("c")
    s = lax.axis_index("s")
    w = c * NS + s

    def iissue(i, b, p):
        off = pl.multiple_of(w * EPW + i * C, C)
        pltpu.async_copy(srcf_hbm.at[pl.ds(off, C)], idx[b][p].at[0],
                         isems[b])
        pltpu.async_copy(dstf_hbm.at[pl.ds(off, C)], idx[b][p].at[1],
                         isems[b])

    def iwait(b):
        pltpu.make_async_copy(srcf_hbm.at[pl.ds(0, C)], idx[b][0].at[0],
                              isems[b]).wait()
        pltpu.make_async_copy(srcf_hbm.at[pl.ds(0, C)], idx[b][0].at[1],
                              isems[b]).wait()

    def gissue(i, b, p):
        pltpu.async_copy(g_hbm.at[idx[b][p].at[0]], rows[b], gsems[b])

    def gwait(b):
        pltpu.make_async_copy(g_hbm.at[idx[0][0].at[0]], rows[b],
                              gsems[b]).wait()

    def sissue(i, b, p):
        pltpu.async_copy(rows[b], acc_sp.at[idx[b][p].at[1]], ssems[b],
                         add=True)

    def swait(b):
        pltpu.make_async_copy(rows[b], acc_sp.at[idx[0][0].at[1]],
                              ssems[b]).wait()

    def service(i):
        b, p = i % NBUF, (i // NBUF) % 2
        b1, p1 = (i + 1) % NBUF, ((i + 1) // NBUF) % 2
        if i + 1 <= LAST:
            iwait(b1)
            if i >= 3:
                swait(b1)
            gissue(i + 1, b1, p1)
        gwait(b)
        sissue(i, b, p)
        if i + 4 <= LAST:
            iissue(i + 4, b, 1 - p)

    for b in range(NBUF):
        iissue(b, b, 0)
    pltpu.sync_copy(zeros_hbm, rows[0])
    for j in range(8):
        sz = C if j < 7 else RPA - 7 * C
        pltpu.async_copy(rows[0].at[pl.ds(0, sz)],
                         acc_sp.at[pl.ds(s * RPA + j * C, sz)], zsem)
    for j in range(8):
        sz = C if j < 7 else RPA - 7 * C
        pltpu.make_async_copy(rows[0].at[pl.ds(0, sz)],
                              acc_sp.at[pl.ds(0, sz)], zsem).wait()
    plsc.subcore_barrier()
    iwait(0)
    gissue(0, 0, 0)
    for i in range(8):
        service(i)

    def body(j, carry):
        i0 = j * 8
        for k in range(8):
            i = i0 + k
            b, p = k % NBUF, (k // NBUF) % 2
            b1 = (k + 1) % NBUF
            p1 = ((k + 1) // NBUF) % 2
            iwait(b1)
            swait(b1)
            gissue(i + 1, b1, p1)
            gwait(b)
            sissue(i, b, p)
            iissue(i + 4, b, 1 - p)
        return carry

    lax.fori_loop(1, 15, body, 0)

    for i in range(120, NCHUNK):
        service(i)
    for b in [1, 2, 3, 0]:
        swait(b)
    plsc.subcore_barrier()

    def dread(j, b):
        sz = C if j < 7 else RPA - 7 * C
        pltpu.async_copy(acc_sp.at[pl.ds(s * RPA + j * C, sz)],
                         rows[b].at[pl.ds(0, sz)], gsems[b])

    def dwrite(j, b):
        sz = C if j < 7 else RPA - 7 * C
        pltpu.make_async_copy(acc_sp.at[pl.ds(0, sz)],
                              rows[b].at[pl.ds(0, sz)], gsems[b]).wait()
        @pl.when(c == 0)
        def _():
            pltpu.async_copy(rows[b].at[pl.ds(0, sz)],
                             out0_hbm.at[pl.ds(s * RPA + j * C, sz)],
                             ssems[b])
        @pl.when(c == 1)
        def _():
            pltpu.async_copy(rows[b].at[pl.ds(0, sz)],
                             out1_hbm.at[pl.ds(s * RPA + j * C, sz)],
                             ssems[b])

    def dwwait(j, b):
        sz = C if j < 7 else RPA - 7 * C
        pltpu.make_async_copy(rows[b].at[pl.ds(0, sz)],
                              out0_hbm.at[pl.ds(0, sz)], ssems[b]).wait()

    dread(0, 0)
    dread(1, 1)
    for j in range(8):
        b = j % 2
        dwrite(j, b)
        if j + 2 < 8:
            dwwait(j, b)
            dread(j + 2, b)
    dwwait(6, 0)
    dwwait(7, 1)


_BLK = 5120
_GRID = NP // _BLK


def _row_spec():
    return pl.BlockSpec((_BLK, D), lambda i: (i, 0))


def _col_spec():
    return pl.BlockSpec((_BLK, 1), lambda i: (i, 0))


def _full_spec():
    return pl.BlockSpec((D, D), lambda i: (0, 0))


def _bias_spec():
    return pl.BlockSpec((1, D), lambda i: (0, 0))


def _tc1_body(d0_ref, d1_ref, x_ref, w1_ref, g_ref, dinv_ref):
    deg = d0_ref[...] + d1_ref[...] - 1.0
    dinv = lax.rsqrt(deg)
    h = jnp.dot(x_ref[...], w1_ref[...], preferred_element_type=jnp.float32)
    g_ref[...] = h * dinv
    dinv_ref[...] = dinv


_tc1 = pl.pallas_call(
    _tc1_body,
    grid=(_GRID,),
    in_specs=[_col_spec(), _col_spec(), _row_spec(), _full_spec()],
    out_specs=[_row_spec(), _col_spec()],
    out_shape=(jax.ShapeDtypeStruct((NP, D), jnp.float32),
               jax.ShapeDtypeStruct((NP, 1), jnp.float32)),
)


def _tc2_body(p0_ref, p1_ref, g1_ref, dinv_ref, b1_ref, w2_ref, g2_ref):
    dinv = dinv_ref[...]
    a = dinv * (p0_ref[...] + p1_ref[...] + g1_ref[...]) + b1_ref[...]
    t = 0.5 * a * (1.0 + lax.erf(a * 0.7071067811865476))
    g2_ref[...] = jnp.dot(t, w2_ref[...],
                          preferred_element_type=jnp.float32) * dinv


_tc2 = pl.pallas_call(
    _tc2_body,
    grid=(_GRID,),
    in_specs=[_row_spec(), _row_spec(), _row_spec(), _col_spec(),
              _bias_spec(), _full_spec()],
    out_specs=_row_spec(),
    out_shape=jax.ShapeDtypeStruct((NP, D), jnp.float32),
)


def _tc3_body(q0_ref, q1_ref, g2_ref, dinv_ref, b2_ref, out_ref):
    out_ref[...] = (dinv_ref[...] * (q0_ref[...] + q1_ref[...] + g2_ref[...])
                    + b2_ref[...])


_tc3 = pl.pallas_call(
    _tc3_body,
    grid=(_GRID,),
    in_specs=[_row_spec(), _row_spec(), _row_spec(), _col_spec(),
              _bias_spec()],
    out_specs=_row_spec(),
    out_shape=jax.ShapeDtypeStruct((N, D), jnp.float32),
)


def kernel(x, edge_index, W1, b1, W2, b2):
    dst_r = edge_index[1].reshape(NW, NCHUNK, C)
    src_f = edge_index[0]
    dst_f = edge_index[1]
    ones_c = jnp.ones((RPS,), jnp.float32)
    zeros_c = jnp.zeros((C, D), jnp.float32)

    d0, d1 = _deg_kernel(dst_r, ones_c)
    g1, dinv = _tc1(d0.reshape(NP, 1), d1.reshape(NP, 1), x, W1)
    p0, p1 = _agg_kernel(g1, src_f, dst_f, zeros_c)
    g2 = _tc2(p0, p1, g1, dinv, b1.reshape(1, D), W2)
    q0, q1 = _agg_kernel(g2, src_f, dst_f, zeros_c)
    return _tc3(q0, q1, g2, dinv, b2.reshape(1, D))
